# baseline (speedup 1.0000x reference)
import functools

import jax
import jax.numpy as jnp
from jax import lax
from jax.experimental import pallas as pl
from jax.experimental.pallas import tpu as pltpu
from jax.experimental.pallas import tpu_sc as plsc

N_NODES = 10000
D = 128
NC, NS, LANES = 2, 16, 16
CHUNK = 128
NBUF = 2
GRP = 4
N_PAD = 10240
ROWS_PER_TILE = N_PAD // NS
LAG = 1


def _edge_pass_builder(n0: int, n1: int, with_deg: bool):
    assert n0 % GRP == 0 and n1 % GRP == 0

    out_types = [jax.ShapeDtypeStruct((NC, N_PAD, D), jnp.float32)]
    if with_deg:
        out_types.append(jax.ShapeDtypeStruct((NC, N_PAD), jnp.float32))

    scratch = [pltpu.VMEM((CHUNK, D), jnp.float32) for _ in range(NBUF)]
    scratch += [pltpu.VMEM((GRP, CHUNK), jnp.int32) for _ in range(4)]
    scratch += [pltpu.VMEM((ROWS_PER_TILE,), jnp.float32)]
    if with_deg:
        scratch += [pltpu.VMEM((CHUNK,), jnp.float32)]
    scratch += [pltpu.VMEM_SHARED((N_PAD, D), jnp.float32)]
    if with_deg:
        scratch += [pltpu.VMEM_SHARED((N_PAD,), jnp.float32)]
    scratch += [pltpu.SemaphoreType.DMA for _ in range(2 * NBUF + 3)]

    def body(*refs):
        n_in = 3 + (1 if with_deg else 0)
        x_hbm, src_hbm, dst_hbm = refs[:3]
        r = refs[n_in:]
        if with_deg:
            ones_hbm = refs[3]
            out_hbm, deg_hbm = r[0], r[1]
            r = r[2:]
        else:
            out_hbm = r[0]
            r = r[1:]
        rows = r[0:NBUF]
        srcb = r[NBUF:NBUF + 2]
        dstb = r[NBUF + 2:NBUF + 4]
        zcol_v = r[NBUF + 4]
        i = NBUF + 5
        if with_deg:
            ones_v = r[i]
            i += 1
        acc_sh = r[i]
        i += 1
        if with_deg:
            deg_sh = r[i]
            i += 1
        sems = r[i:i + NBUF]
        ssems = r[i + NBUF:i + 2 * NBUF]
        isems = r[i + 2 * NBUF:i + 2 * NBUF + 2]
        dsem = r[i + 2 * NBUF + 2]

        c = lax.axis_index("c")
        s = lax.axis_index("s")
        row_base = jnp.where(c == 0, s * n0, NS * n0 + s * n1)
        base = s * ROWS_PER_TILE

        def stage(g):
            p = g % 2
            pltpu.async_copy(src_hbm.at[pl.ds(row_base + g * GRP, GRP)],
                             srcb[p], isems[0])
            pltpu.async_copy(dst_hbm.at[pl.ds(row_base + g * GRP, GRP)],
                             dstb[p], isems[1])

        def stage_wait(g):
            p = g % 2
            pltpu.make_async_copy(src_hbm.at[pl.ds(row_base + g * GRP, GRP)],
                                  srcb[p], isems[0]).wait()
            pltpu.make_async_copy(dst_hbm.at[pl.ds(row_base + g * GRP, GRP)],
                                  dstb[p], isems[1]).wait()

        stage(0)
        if with_deg:
            pltpu.sync_copy(ones_hbm, ones_v)
        zv = jnp.zeros((LANES,), jnp.float32)

        def _zrow(rr, _):
            for k in range(D // LANES):
                rows[0][rr, pl.ds(k * LANES, LANES)] = zv
            return 0

        lax.fori_loop(0, CHUNK, _zrow, 0)

        def _zcol(rr, _):
            zcol_v[pl.ds(rr * LANES, LANES)] = zv
            return 0

        lax.fori_loop(0, ROWS_PER_TILE // LANES, _zcol, 0)
        for k in range(ROWS_PER_TILE // CHUNK):
            pltpu.sync_copy(rows[0], acc_sh.at[pl.ds(base + k * CHUNK, CHUNK)])
        if with_deg:
            pltpu.sync_copy(zcol_v, deg_sh.at[pl.ds(base, ROWS_PER_TILE)])
        stage_wait(0)
        stage(1)
        plsc.subcore_barrier()

        def srow(j):
            return (j // GRP) % 2, j % GRP

        def g_start(j):
            p, jj = srow(j)
            pltpu.async_copy(x_hbm.at[srcb[p].at[jj]], rows[j % NBUF],
                             sems[j % NBUF])

        def g_wait(j):
            p, jj = srow(j)
            pltpu.make_async_copy(x_hbm.at[srcb[p].at[jj]], rows[j % NBUF],
                                  sems[j % NBUF]).wait()

        def s_start(j):
            p, jj = srow(j)
            pltpu.async_copy(rows[j % NBUF], acc_sh.at[dstb[p].at[jj]],
                             ssems[j % NBUF], add=True)
            if with_deg:
                pltpu.async_copy(ones_v, deg_sh.at[dstb[p].at[jj]], dsem,
                                 add=True)

        def s_wait(j):
            p, jj = srow(j)
            pltpu.make_async_copy(rows[j % NBUF], acc_sh.at[dstb[p].at[jj]],
                                  ssems[j % NBUF]).wait()
            if with_deg:
                pltpu.make_async_copy(ones_v, deg_sh.at[dstb[p].at[jj]],
                                      dsem).wait()

        def emit(n_chunks):
            n_grps = n_chunks // GRP
            for j in range(min(LAG, n_chunks)):
                g_start(j)
            for j in range(n_chunks):
                g, jj = j // GRP, j % GRP
                if jj == LAG and 0 < g and g + 1 < n_grps:
                    stage(g + 1)
                if jj == GRP - LAG and g + 1 < n_grps:
                    stage_wait(g + 1)
                g_wait(j)
                s_start(j)
                if j - LAG >= 0:
                    s_wait(j - LAG)
                if j + LAG < n_chunks:
                    g_start(j + LAG)
            for j in range(max(0, n_chunks - LAG), n_chunks):
                s_wait(j)

        if n0 == n1:
            emit(n0)
        else:
            @pl.when(c == 0)
            def _():
                emit(n0)

            @pl.when(c == 1)
            def _():
                emit(n1)

        plsc.subcore_barrier()
        pltpu.sync_copy(acc_sh.at[pl.ds(base, ROWS_PER_TILE)],
                        out_hbm.at[c, pl.ds(base, ROWS_PER_TILE)])
        if with_deg:
            pltpu.sync_copy(deg_sh.at[pl.ds(base, ROWS_PER_TILE)],
                            deg_hbm.at[c, pl.ds(base, ROWS_PER_TILE)])

    mesh = plsc.VectorSubcoreMesh(core_axis_name="c", subcore_axis_name="s")
    return pl.kernel(body, mesh=mesh, out_type=out_types, scratch_types=scratch)


BR = 1024


def _mid_body(aggp, degp, x, wrel, brel, wroot, wgcn, bgcn, g_o, st_o, dinv_o):
    agg = aggp[0] + aggp[1]
    deg = degp[0] + degp[1] + 1.0
    dinv = 1.0 / jnp.sqrt(deg)
    h = jnp.dot(agg, wrel[...], preferred_element_type=jnp.float32)
    h = h + brel[...][None, :]
    h = h + jnp.dot(x[...], wroot[...], preferred_element_type=jnp.float32)
    h = jnp.maximum(h, 0.0)
    h2 = jnp.dot(h, wgcn[...], preferred_element_type=jnp.float32)
    g_o[...] = dinv[:, None] * h2
    st_o[...] = (dinv * dinv)[:, None] * h2 + bgcn[...][None, :]
    dinv_o[...] = dinv


def _fin_body(qp, dinv, st, out):
    q = qp[0] + qp[1]
    t = dinv[...][:, None] * q + st[...]
    m = jnp.max(t, axis=1, keepdims=True)
    lse = jnp.log(jnp.sum(jnp.exp(t - m), axis=1, keepdims=True)) + m
    out[...] = t - lse


def kernel(x, edge_index, W_rel, b_rel, W_root, W_gcn, b_gcn):
    E = edge_index.shape[1]
    src = edge_index[0].astype(jnp.int32)
    dst = edge_index[1].astype(jnp.int32)

    n_chunks = -(-E // (NC * NS * CHUNK))
    n_chunks = -(-n_chunks // GRP) * GRP
    n0 = min(2 * n_chunks - GRP, int(round(2 * n_chunks * 0.875 / GRP)) * GRP)
    n1 = 2 * n_chunks - n0
    e_pad = (n0 + n1) * NS * CHUNK
    pad = e_pad - E
    src_p = jnp.concatenate([src, jnp.zeros((pad,), jnp.int32)])
    dst_p = jnp.concatenate([dst, jnp.full((pad,), N_NODES, jnp.int32)])
    src_p = src_p.reshape(NS * (n0 + n1), CHUNK)
    dst_p = dst_p.reshape(NS * (n0 + n1), CHUNK)

    ones = jnp.ones((CHUNK,), jnp.float32)

    pass1 = _edge_pass_builder(n0, n1, with_deg=True)
    agg_p, deg_p = pass1(x, src_p, dst_p, ones)

    grid = N_PAD // BR
    x_pad = jnp.pad(x, ((0, N_PAD - N_NODES), (0, 0)))
    g, st, dinv = pl.pallas_call(
        _mid_body,
        grid=(grid,),
        in_specs=[
            pl.BlockSpec((2, BR, D), lambda i: (0, i, 0)),
            pl.BlockSpec((2, BR), lambda i: (0, i)),
            pl.BlockSpec((BR, D), lambda i: (i, 0)),
            pl.BlockSpec((D, D), lambda i: (0, 0)),
            pl.BlockSpec((D,), lambda i: (0,)),
            pl.BlockSpec((D, D), lambda i: (0, 0)),
            pl.BlockSpec((D, D), lambda i: (0, 0)),
            pl.BlockSpec((D,), lambda i: (0,)),
        ],
        out_specs=[
            pl.BlockSpec((BR, D), lambda i: (i, 0)),
            pl.BlockSpec((BR, D), lambda i: (i, 0)),
            pl.BlockSpec((BR,), lambda i: (i,)),
        ],
        out_shape=[
            jax.ShapeDtypeStruct((N_PAD, D), jnp.float32),
            jax.ShapeDtypeStruct((N_PAD, D), jnp.float32),
            jax.ShapeDtypeStruct((N_PAD,), jnp.float32),
        ],
    )(agg_p, deg_p, x_pad, W_rel, b_rel, W_root, W_gcn, b_gcn)

    pass2 = _edge_pass_builder(n0, n1, with_deg=False)
    res2 = pass2(g, src_p, dst_p)
    q_p = res2[0] if isinstance(res2, (list, tuple)) else res2

    out = pl.pallas_call(
        _fin_body,
        grid=(grid,),
        in_specs=[
            pl.BlockSpec((2, BR, D), lambda i: (0, i, 0)),
            pl.BlockSpec((BR,), lambda i: (i,)),
            pl.BlockSpec((BR, D), lambda i: (i, 0)),
        ],
        out_specs=pl.BlockSpec((BR, D), lambda i: (i, 0)),
        out_shape=jax.ShapeDtypeStruct((N_PAD, D), jnp.float32),
    )(q_p, dinv, st)
    return out[:N_NODES]


# speedup vs baseline: 1.0327x; 1.0327x over previous
import functools

import jax
import jax.numpy as jnp
from jax import lax
from jax.experimental import pallas as pl
from jax.experimental.pallas import tpu as pltpu
from jax.experimental.pallas import tpu_sc as plsc

N_NODES = 10000
D = 128
NC, NS, LANES = 2, 16, 16
CHUNK = 64
NBUF = 4
GRP = 8
N_PAD = 10240
ROWS_PER_TILE = N_PAD // NS
LAG = 3


def _edge_pass_builder(n0: int, n1: int, with_deg: bool):
    assert n0 % GRP == 0 and n1 % GRP == 0

    out_types = [jax.ShapeDtypeStruct((NC, N_PAD, D), jnp.float32)]
    if with_deg:
        out_types.append(jax.ShapeDtypeStruct((NC, N_PAD), jnp.float32))

    scratch = [pltpu.VMEM((CHUNK, D), jnp.float32) for _ in range(NBUF)]
    scratch += [pltpu.VMEM((GRP, CHUNK), jnp.int32) for _ in range(4)]
    scratch += [pltpu.VMEM((ROWS_PER_TILE,), jnp.float32)]
    if with_deg:
        scratch += [pltpu.VMEM((CHUNK,), jnp.float32)]
    scratch += [pltpu.VMEM_SHARED((N_PAD, D), jnp.float32)]
    if with_deg:
        scratch += [pltpu.VMEM_SHARED((N_PAD,), jnp.float32)]
    scratch += [pltpu.SemaphoreType.DMA for _ in range(2 * NBUF + 3)]

    def body(*refs):
        n_in = 3 + (1 if with_deg else 0)
        x_hbm, src_hbm, dst_hbm = refs[:3]
        r = refs[n_in:]
        if with_deg:
            ones_hbm = refs[3]
            out_hbm, deg_hbm = r[0], r[1]
            r = r[2:]
        else:
            out_hbm = r[0]
            r = r[1:]
        rows = r[0:NBUF]
        srcb = r[NBUF:NBUF + 2]
        dstb = r[NBUF + 2:NBUF + 4]
        zcol_v = r[NBUF + 4]
        i = NBUF + 5
        if with_deg:
            ones_v = r[i]
            i += 1
        acc_sh = r[i]
        i += 1
        if with_deg:
            deg_sh = r[i]
            i += 1
        sems = r[i:i + NBUF]
        ssems = r[i + NBUF:i + 2 * NBUF]
        isems = r[i + 2 * NBUF:i + 2 * NBUF + 2]
        dsem = r[i + 2 * NBUF + 2]

        c = lax.axis_index("c")
        s = lax.axis_index("s")
        row_base = jnp.where(c == 0, s * n0, NS * n0 + s * n1)
        base = s * ROWS_PER_TILE

        def stage(g):
            p = g % 2
            pltpu.async_copy(src_hbm.at[pl.ds(row_base + g * GRP, GRP)],
                             srcb[p], isems[0])
            pltpu.async_copy(dst_hbm.at[pl.ds(row_base + g * GRP, GRP)],
                             dstb[p], isems[1])

        def stage_wait(g):
            p = g % 2
            pltpu.make_async_copy(src_hbm.at[pl.ds(row_base + g * GRP, GRP)],
                                  srcb[p], isems[0]).wait()
            pltpu.make_async_copy(dst_hbm.at[pl.ds(row_base + g * GRP, GRP)],
                                  dstb[p], isems[1]).wait()

        stage(0)
        if with_deg:
            pltpu.sync_copy(ones_hbm, ones_v)
        zv = jnp.zeros((LANES,), jnp.float32)

        def _zrow(rr, _):
            for k in range(D // LANES):
                rows[0][rr, pl.ds(k * LANES, LANES)] = zv
            return 0

        lax.fori_loop(0, CHUNK, _zrow, 0)

        def _zcol(rr, _):
            zcol_v[pl.ds(rr * LANES, LANES)] = zv
            return 0

        lax.fori_loop(0, ROWS_PER_TILE // LANES, _zcol, 0)
        for k in range(ROWS_PER_TILE // CHUNK):
            pltpu.sync_copy(rows[0], acc_sh.at[pl.ds(base + k * CHUNK, CHUNK)])
        if with_deg:
            pltpu.sync_copy(zcol_v, deg_sh.at[pl.ds(base, ROWS_PER_TILE)])
        stage_wait(0)
        stage(1)
        plsc.subcore_barrier()

        def srow(j):
            return (j // GRP) % 2, j % GRP

        def g_start(j):
            p, jj = srow(j)
            pltpu.async_copy(x_hbm.at[srcb[p].at[jj]], rows[j % NBUF],
                             sems[j % NBUF])

        def g_wait(j):
            p, jj = srow(j)
            pltpu.make_async_copy(x_hbm.at[srcb[p].at[jj]], rows[j % NBUF],
                                  sems[j % NBUF]).wait()

        def s_start(j):
            p, jj = srow(j)
            pltpu.async_copy(rows[j % NBUF], acc_sh.at[dstb[p].at[jj]],
                             ssems[j % NBUF], add=True)
            if with_deg:
                pltpu.async_copy(ones_v, deg_sh.at[dstb[p].at[jj]], dsem,
                                 add=True)

        def s_wait(j):
            p, jj = srow(j)
            pltpu.make_async_copy(rows[j % NBUF], acc_sh.at[dstb[p].at[jj]],
                                  ssems[j % NBUF]).wait()
            if with_deg:
                pltpu.make_async_copy(ones_v, deg_sh.at[dstb[p].at[jj]],
                                      dsem).wait()

        def emit(n_chunks):
            n_grps = n_chunks // GRP
            for j in range(min(LAG, n_chunks)):
                g_start(j)
            for j in range(n_chunks):
                g, jj = j // GRP, j % GRP
                if jj == LAG and 0 < g and g + 1 < n_grps:
                    stage(g + 1)
                if jj == GRP - LAG and g + 1 < n_grps:
                    stage_wait(g + 1)
                g_wait(j)
                s_start(j)
                if j - LAG >= 0:
                    s_wait(j - LAG)
                if j + LAG < n_chunks:
                    g_start(j + LAG)
            for j in range(max(0, n_chunks - LAG), n_chunks):
                s_wait(j)

        if n0 == n1:
            emit(n0)
        else:
            @pl.when(c == 0)
            def _():
                emit(n0)

            @pl.when(c == 1)
            def _():
                emit(n1)

        plsc.subcore_barrier()
        pltpu.sync_copy(acc_sh.at[pl.ds(base, ROWS_PER_TILE)],
                        out_hbm.at[c, pl.ds(base, ROWS_PER_TILE)])
        if with_deg:
            pltpu.sync_copy(deg_sh.at[pl.ds(base, ROWS_PER_TILE)],
                            deg_hbm.at[c, pl.ds(base, ROWS_PER_TILE)])

    mesh = plsc.VectorSubcoreMesh(core_axis_name="c", subcore_axis_name="s")
    return pl.kernel(body, mesh=mesh, out_type=out_types, scratch_types=scratch)


BR = 1024


def _mid_body(aggp, degp, x, wrel, brel, wroot, wgcn, bgcn, g_o, st_o, dinv_o):
    agg = aggp[0] + aggp[1]
    deg = degp[0] + degp[1] + 1.0
    dinv = 1.0 / jnp.sqrt(deg)
    h = jnp.dot(agg, wrel[...], preferred_element_type=jnp.float32)
    h = h + brel[...][None, :]
    h = h + jnp.dot(x[...], wroot[...], preferred_element_type=jnp.float32)
    h = jnp.maximum(h, 0.0)
    h2 = jnp.dot(h, wgcn[...], preferred_element_type=jnp.float32)
    g_o[...] = dinv[:, None] * h2
    st_o[...] = (dinv * dinv)[:, None] * h2 + bgcn[...][None, :]
    dinv_o[...] = dinv


def _fin_body(qp, dinv, st, out):
    q = qp[0] + qp[1]
    t = dinv[...][:, None] * q + st[...]
    m = jnp.max(t, axis=1, keepdims=True)
    lse = jnp.log(jnp.sum(jnp.exp(t - m), axis=1, keepdims=True)) + m
    out[...] = t - lse


def kernel(x, edge_index, W_rel, b_rel, W_root, W_gcn, b_gcn):
    E = edge_index.shape[1]
    src = edge_index[0].astype(jnp.int32)
    dst = edge_index[1].astype(jnp.int32)

    n_chunks = -(-E // (NC * NS * CHUNK))
    n_chunks = -(-n_chunks // GRP) * GRP
    n0 = min(2 * n_chunks - GRP, int(round(2 * n_chunks * 0.875 / GRP)) * GRP)
    n1 = 2 * n_chunks - n0
    e_pad = (n0 + n1) * NS * CHUNK
    pad = e_pad - E
    src_p = jnp.concatenate([src, jnp.zeros((pad,), jnp.int32)])
    dst_p = jnp.concatenate([dst, jnp.full((pad,), N_NODES, jnp.int32)])
    src_p = src_p.reshape(NS * (n0 + n1), CHUNK)
    dst_p = dst_p.reshape(NS * (n0 + n1), CHUNK)

    ones = jnp.ones((CHUNK,), jnp.float32)

    pass1 = _edge_pass_builder(n0, n1, with_deg=True)
    agg_p, deg_p = pass1(x, src_p, dst_p, ones)

    grid = N_PAD // BR
    x_pad = jnp.pad(x, ((0, N_PAD - N_NODES), (0, 0)))
    g, st, dinv = pl.pallas_call(
        _mid_body,
        grid=(grid,),
        in_specs=[
            pl.BlockSpec((2, BR, D), lambda i: (0, i, 0)),
            pl.BlockSpec((2, BR), lambda i: (0, i)),
            pl.BlockSpec((BR, D), lambda i: (i, 0)),
            pl.BlockSpec((D, D), lambda i: (0, 0)),
            pl.BlockSpec((D,), lambda i: (0,)),
            pl.BlockSpec((D, D), lambda i: (0, 0)),
            pl.BlockSpec((D, D), lambda i: (0, 0)),
            pl.BlockSpec((D,), lambda i: (0,)),
        ],
        out_specs=[
            pl.BlockSpec((BR, D), lambda i: (i, 0)),
            pl.BlockSpec((BR, D), lambda i: (i, 0)),
            pl.BlockSpec((BR,), lambda i: (i,)),
        ],
        out_shape=[
            jax.ShapeDtypeStruct((N_PAD, D), jnp.float32),
            jax.ShapeDtypeStruct((N_PAD, D), jnp.float32),
            jax.ShapeDtypeStruct((N_PAD,), jnp.float32),
        ],
    )(agg_p, deg_p, x_pad, W_rel, b_rel, W_root, W_gcn, b_gcn)

    pass2 = _edge_pass_builder(n0, n1, with_deg=False)
    res2 = pass2(g, src_p, dst_p)
    q_p = res2[0] if isinstance(res2, (list, tuple)) else res2

    out = pl.pallas_call(
        _fin_body,
        grid=(grid,),
        in_specs=[
            pl.BlockSpec((2, BR, D), lambda i: (0, i, 0)),
            pl.BlockSpec((BR,), lambda i: (i,)),
            pl.BlockSpec((BR, D), lambda i: (i, 0)),
        ],
        out_specs=pl.BlockSpec((BR, D), lambda i: (i, 0)),
        out_shape=jax.ShapeDtypeStruct((N_PAD, D), jnp.float32),
    )(q_p, dinv, st)
    return out[:N_NODES]
